# baseline (speedup 1.0000x reference)
import dataclasses
import functools

import jax
import jax.numpy as jnp
from jax.experimental import pallas as pl
from jax.experimental.pallas import tpu as pltpu
from jax.experimental.pallas import tpu_sc as plsc

N, D = 16384, 128
L = 16
TILE_R = 64
TILE_E = TILE_R * D

_mesh = plsc.VectorSubcoreMesh(core_axis_name="c", subcore_axis_name="s")

_cp = pltpu.CompilerParams()
if "needs_layout_passes" in pltpu.CompilerParams.__dataclass_fields__:
    _cp = dataclasses.replace(_cp, needs_layout_passes=False)


@functools.partial(
    pl.kernel,
    out_type=[
        jax.ShapeDtypeStruct((N * D,), jnp.float32),
        jax.ShapeDtypeStruct((N,), jnp.float32),
    ],
    mesh=_mesh,
    compiler_params=_cp,
    scratch_types=[
        pltpu.VMEM((L,), jnp.float32),
        pltpu.VMEM((L,), jnp.float32),
        pltpu.VMEM((L,), jnp.float32),
    ],
)
def _sc_transform(in_hbm, ctx_hbm, sv_hbm, bv_hbm, lv_hbm, out_hbm, ld_hbm,
                  sv_v, bv_v, lv_v):
    pltpu.sync_copy(sv_hbm, sv_v)
    pltpu.sync_copy(bv_hbm, bv_v)
    pltpu.sync_copy(lv_hbm, lv_v)
    zero = jnp.zeros((L,), jnp.float32)
    lane = jnp.arange(L, dtype=jnp.int32)

    def body(in_t, ctx_t, out_t, ld_t):
        sv = sv_v[...]
        bv = bv_v[...]
        lv = lv_v[...]
        for g in range(TILE_R // L):
            merged = zero
            for j in range(L):
                acc = zero
                for c in range(D // L):
                    off = (g * L + j) * D + c * L
                    x = in_t[pl.ds(off, L)]
                    t = ctx_t[pl.ds(off, L)]
                    m = t > 0.0
                    out_t[pl.ds(off, L)] = jnp.where(m, x * sv + bv, x)
                    acc = acc + jnp.where(m, lv, zero)
                merged = jnp.where(lane == j, jnp.sum(acc), merged)
            ld_t[pl.ds(g * L, L)] = merged

    pltpu.emit_pipeline(
        body,
        grid=(N // TILE_R,),
        in_specs=[
            pl.BlockSpec((TILE_E,), lambda i: (i,)),
            pl.BlockSpec((TILE_E,), lambda i: (i,)),
        ],
        out_specs=[
            pl.BlockSpec((TILE_E,), lambda i: (i,)),
            pl.BlockSpec((TILE_R,), lambda i: (i,)),
        ],
        core_axis_name=("c", "s"),
        dimension_semantics=(pltpu.PARALLEL,),
    )(in_hbm, ctx_hbm, out_hbm, ld_hbm)


def kernel(inputs, context, log_scale, shift):
    sv = jnp.broadcast_to(jnp.exp(log_scale), (L,))
    bv = jnp.broadcast_to(shift, (L,))
    lv = jnp.broadcast_to(log_scale, (L,))
    out_flat, logabsdet = _sc_transform(
        inputs.reshape(N * D), context.reshape(N * D), sv, bv, lv)
    return out_flat.reshape(N, D), logabsdet


# speedup vs baseline: 2.7986x; 2.7986x over previous
import dataclasses
import functools

import jax
import jax.numpy as jnp
from jax.experimental import pallas as pl
from jax.experimental.pallas import tpu as pltpu
from jax.experimental.pallas import tpu_sc as plsc

N, D = 16384, 128
L = 16
TILE_R = 128
TILE_E = TILE_R * D
TC_BLOCK_R = 2048

_mesh = plsc.VectorSubcoreMesh(core_axis_name="c", subcore_axis_name="s")

_cp = pltpu.CompilerParams()
if "needs_layout_passes" in pltpu.CompilerParams.__dataclass_fields__:
    _cp = dataclasses.replace(_cp, needs_layout_passes=False)


@functools.partial(
    pl.kernel,
    out_type=jax.ShapeDtypeStruct((N,), jnp.float32),
    mesh=_mesh,
    compiler_params=_cp,
    scratch_types=[pltpu.VMEM((L,), jnp.float32)],
)
def _sc_logabsdet(ctx_hbm, lv_hbm, ld_hbm, lv_v):
    pltpu.sync_copy(lv_hbm, lv_v)
    zero = jnp.zeros((L,), jnp.float32)
    lane = jnp.arange(L, dtype=jnp.int32)

    def body(ctx_t, ld_t):
        lv = lv_v[...]
        for g in range(TILE_R // L):
            merged = zero
            for j in range(L):
                acc = zero
                for c in range(D // L):
                    off = (g * L + j) * D + c * L
                    t = ctx_t[pl.ds(off, L)]
                    acc = acc + jnp.where(t > 0.0, lv, zero)
                merged = jnp.where(lane == j, jnp.sum(acc), merged)
            ld_t[pl.ds(g * L, L)] = merged

    pltpu.emit_pipeline(
        body,
        grid=(N // TILE_R,),
        in_specs=[pl.BlockSpec((TILE_E,), lambda i: (i,))],
        out_specs=[pl.BlockSpec((TILE_R,), lambda i: (i,))],
        core_axis_name=("c", "s"),
        dimension_semantics=(pltpu.PARALLEL,),
    )(ctx_hbm, ld_hbm)


def _tc_body(x_ref, c_ref, s_ref, b_ref, o_ref):
    o_ref[...] = jnp.where(c_ref[...] > 0.0,
                           x_ref[...] * s_ref[0, 0] + b_ref[0, 0],
                           x_ref[...])


_tc_transform = pl.pallas_call(
    _tc_body,
    grid=(N // TC_BLOCK_R,),
    in_specs=[
        pl.BlockSpec((TC_BLOCK_R, D), lambda i: (i, 0)),
        pl.BlockSpec((TC_BLOCK_R, D), lambda i: (i, 0)),
        pl.BlockSpec((1, 1), lambda i: (0, 0)),
        pl.BlockSpec((1, 1), lambda i: (0, 0)),
    ],
    out_specs=pl.BlockSpec((TC_BLOCK_R, D), lambda i: (i, 0)),
    out_shape=jax.ShapeDtypeStruct((N, D), jnp.float32),
)


def kernel(inputs, context, log_scale, shift):
    sv = jnp.exp(log_scale).reshape(1, 1)
    bv = shift.reshape(1, 1)
    lv = jnp.broadcast_to(log_scale, (L,))
    outputs = _tc_transform(inputs, context, sv, bv)
    logabsdet = _sc_logabsdet(context.reshape(N * D), lv)
    return outputs, logabsdet
